# baseline (speedup 1.0000x reference)
import functools

import jax
import jax.numpy as jnp
import numpy as np
from jax import lax
from jax.experimental import pallas as pl
from jax.experimental.pallas import tpu as pltpu
from jax.experimental.pallas import tpu_sc as plsc

D_MODEL = 512
SPD = 86400
PERIODS = [SPD, SPD // 2, SPD // 3, SPD // 4, SPD // 6]
SUB = D_MODEL // len(PERIODS)
NT = len(PERIODS)
GR = 16
SPAN = 8
_MOD_STEPS = [(), (1,), (2, 1), (2, 1), (4, 2, 1)]


def _mod_p(v, i):
    for m in _MOD_STEPS[i]:
        q = jnp.int32(m * PERIODS[i])
        v = jnp.where(v >= q, v - q, v)
    return v


def _pe_table(d_model=D_MODEL, max_len=5000, power_freq=50.0, sample_rate=1.0):
    pe = np.zeros((max_len, d_model), dtype=np.float32)
    pos = np.arange(max_len, dtype=np.float32)
    harmonics = [1, 2, 3, 5, 7]
    hd = d_model // (len(harmonics) * 2)
    for h_idx, h in enumerate(harmonics):
        omega = 2.0 * np.pi * power_freq * h / sample_rate
        start = h_idx * hd * 2
        end = min(start + hd * 2, d_model)
        for i in range(0, end - start, 2):
            ps = i * np.pi / (end - start)
            if start + i < d_model:
                pe[:, start + i] = np.sin(pos * omega + ps)
            if start + i + 1 < d_model:
                pe[:, start + i + 1] = np.cos(pos * omega + ps)
    return pe


_PE = _pe_table()


def _sc_gather_span(ti, tviews):
    (bt,) = ti.shape
    nmax = [tv.shape[0] - 1 for tv in tviews]
    info = plsc.get_sparse_core_info()
    nc, ns = info.num_cores, info.num_subcores
    nw = nc * ns
    npw = bt // nw
    nchk = npw // GR
    assert npw % GR == 0

    mesh = plsc.VectorSubcoreMesh(core_axis_name="c", subcore_axis_name="s")

    @functools.partial(
        pl.kernel,
        mesh=mesh,
        compiler_params=pltpu.CompilerParams(use_tc_tiling_on_sc=False,
                                             needs_layout_passes=False),
        out_type=jax.ShapeDtypeStruct((NT, bt * SPAN, GR), jnp.float32),
        scratch_types=[
            pltpu.VMEM((npw,), jnp.int32),
            pltpu.VMEM((nchk, GR * SPAN), jnp.int32),
            pltpu.VMEM((npw * SPAN, GR), jnp.float32),
            pltpu.SemaphoreType.DMA,
        ],
    )
    def gk(t0, t1, t2, t3, t4, ti_hbm, out_hbm, tv, iv, rows, sem):
        wid = lax.axis_index("s") * nc + lax.axis_index("c")
        base = wid * npw
        pltpu.sync_copy(ti_hbm.at[pl.ds(base, npw)], tv)
        tables = [t0, t1, t2, t3, t4]
        lanes = lax.iota(jnp.int32, GR)
        for i in range(NT):
            def fire(ch, carry, i=i):
                t16 = tv[pl.ds(ch * GR, GR)]
                idx = _mod_p(t16, i)
                g0 = lax.shift_right_logical(idx * jnp.int32(SUB),
                                             jnp.int32(4))
                rowv = jnp.full((GR,), ch, jnp.int32)
                for c in range(SPAN):
                    val = g0 + jnp.int32(c)
                    if c == SPAN - 1:
                        val = jnp.minimum(val, jnp.int32(nmax[i]))
                    plsc.store_scatter(iv, [rowv, lanes * SPAN + c], val)
                pltpu.async_copy(
                    tables[i].at[iv.at[ch]],
                    rows.at[pl.ds(ch * GR * SPAN, GR * SPAN)],
                    sem,
                )
                return carry

            lax.fori_loop(0, nchk, fire, 0)

            def drain(ch, carry, i=i):
                pltpu.make_async_copy(
                    out_hbm.at[i, pl.ds(base * SPAN, GR * SPAN), :],
                    rows.at[pl.ds(0, GR * SPAN)],
                    sem,
                ).wait()
                return carry

            lax.fori_loop(0, nchk, drain, 0)
            pltpu.sync_copy(rows,
                            out_hbm.at[i, pl.ds(base * SPAN, npw * SPAN), :])

    return gk(*tviews, ti)


def _tc_body(x_ref, t_ref, g_ref, pe_ref, we_ref, wt_ref, b_ref, o_ref):
    acc = jnp.dot(x_ref[...], we_ref[...], preferred_element_type=jnp.float32)
    acc += pe_ref[...] + b_ref[...]
    tvec = t_ref[...]
    for i in range(NT):
        idx = _mod_p(tvec, i)
        sh = (idx * jnp.int32(6)) & jnp.int32(15)
        span = g_ref[i]
        aligned = jnp.zeros((span.shape[0], SUB), jnp.float32)
        for s in range(0, 16, 2):
            aligned = aligned + jnp.where(sh == s, span[:, s:s + SUB], 0.0)
        acc += jnp.dot(aligned, wt_ref[i], preferred_element_type=jnp.float32)
    o_ref[...] = acc


def _tc_combine(xf, tif, g, pe, we, wt, bias, tb):
    bt, c = xf.shape
    t = pe.shape[0]
    jblocks = t // tb
    b = bt // t
    grid = (jblocks, b)
    w = GR * SPAN
    return pl.pallas_call(
        _tc_body,
        grid=grid,
        in_specs=[
            pl.BlockSpec((tb, c), lambda j, bb: (bb * jblocks + j, 0)),
            pl.BlockSpec((tb, 1), lambda j, bb: (bb * jblocks + j, 0)),
            pl.BlockSpec((NT, tb, w), lambda j, bb: (0, bb * jblocks + j, 0)),
            pl.BlockSpec((tb, D_MODEL), lambda j, bb: (j, 0)),
            pl.BlockSpec((c, D_MODEL), lambda j, bb: (0, 0)),
            pl.BlockSpec((NT, SUB, D_MODEL), lambda j, bb: (0, 0, 0)),
            pl.BlockSpec((1, D_MODEL), lambda j, bb: (0, 0)),
        ],
        out_specs=pl.BlockSpec((tb, D_MODEL), lambda j, bb: (bb * jblocks + j, 0)),
        out_shape=jax.ShapeDtypeStruct((bt, D_MODEL), jnp.float32),
    )(xf, tif, g, pe, we, wt, bias)


def kernel(x, time_indices, value_W, value_b, daily_tab0, daily_tab1,
           daily_tab2, daily_tab3, daily_tab4, daily_W, daily_b, phase_embed,
           pos_W, pos_b, neg_W, neg_b, vq_W, vq_b, vq_cW, vq_cb):
    B, T, C = x.shape
    bt = B * T
    ti = time_indices.reshape(bt).astype(jnp.int32)
    tabs = [daily_tab0, daily_tab1, daily_tab2, daily_tab3, daily_tab4]
    tviews = [t.reshape(-1, GR) for t in tabs]

    g = _sc_gather_span(ti, tviews)
    g = g.reshape(NT, bt, GR * SPAN)

    dq = vq_W.shape[0]
    w_q = vq_cW[:, :dq] @ vq_W[:, 0]
    b_q = vq_cW[:, :dq] @ vq_b + vq_cb
    neg_perm = jnp.stack([neg_W[:, 0], neg_W[:, 2], neg_W[:, 1]], axis=1)
    w_eff = value_W + pos_W + 0.1 * neg_perm + (w_q / 660.0)[:, None]
    bias = (value_b + daily_b + pos_b + 0.1 * neg_b + phase_embed.mean(0)
            + b_q - w_q)

    pe = jnp.asarray(_PE[:T])
    wt = daily_W.T.reshape(NT, SUB, D_MODEL)

    out = _tc_combine(x.reshape(bt, C), ti.reshape(bt, 1), g, pe, w_eff.T, wt,
                      bias.reshape(1, D_MODEL), tb=512)
    return out.reshape(B, T, D_MODEL)


# speedup vs baseline: 1.0184x; 1.0184x over previous
import functools

import jax
import jax.numpy as jnp
import numpy as np
from jax import lax
from jax.experimental import pallas as pl
from jax.experimental.pallas import tpu as pltpu
from jax.experimental.pallas import tpu_sc as plsc

D_MODEL = 512
SPD = 86400
PERIODS = [SPD, SPD // 2, SPD // 3, SPD // 4, SPD // 6]
SUB = D_MODEL // len(PERIODS)
_MOD_STEPS = [(), (1,), (2, 1), (2, 1), (4, 2, 1)]


def _pe_table(d_model=D_MODEL, max_len=5000, power_freq=50.0, sample_rate=1.0):
    pe = np.zeros((max_len, d_model), dtype=np.float32)
    pos = np.arange(max_len, dtype=np.float32)
    harmonics = [1, 2, 3, 5, 7]
    hd = d_model // (len(harmonics) * 2)
    for h_idx, h in enumerate(harmonics):
        omega = 2.0 * np.pi * power_freq * h / sample_rate
        start = h_idx * hd * 2
        end = min(start + hd * 2, d_model)
        for i in range(0, end - start, 2):
            ps = i * np.pi / (end - start)
            if start + i < d_model:
                pe[:, start + i] = np.sin(pos * omega + ps)
            if start + i + 1 < d_model:
                pe[:, start + i + 1] = np.cos(pos * omega + ps)
    return pe


_PE = _pe_table()


WPAD = 128


def _sc_gather(ti, tabs):
    (bt,) = ti.shape
    info = plsc.get_sparse_core_info()
    nc, ns = info.num_cores, info.num_subcores
    nw = nc * ns
    npw = bt // nw
    nch = npw // 128
    assert npw % 128 == 0

    mesh = plsc.VectorSubcoreMesh(core_axis_name="c", subcore_axis_name="s")

    @functools.partial(
        pl.kernel,
        mesh=mesh,
        compiler_params=pltpu.CompilerParams(use_tc_tiling_on_sc=False),
        out_type=jax.ShapeDtypeStruct((len(tabs), bt, WPAD), jnp.bfloat16),
        scratch_types=[
            pltpu.VMEM((npw,), jnp.int32),
            pltpu.VMEM((nch, 128), jnp.int32),
            pltpu.VMEM((npw, WPAD), jnp.bfloat16),
            pltpu.SemaphoreType.DMA,
        ],
    )
    def gk(t0, t1, t2, t3, t4, ti_hbm, out_hbm, tv, iv, rows, sem):
        wid = lax.axis_index("s") * nc + lax.axis_index("c")
        base = wid * npw
        pltpu.sync_copy(ti_hbm.at[pl.ds(base, npw)], tv)
        tables = [t0, t1, t2, t3, t4]
        for i in range(len(tables)):
            p = PERIODS[i]
            for c in range(nch):
                for k in range(128 // 16):
                    v = tv[pl.ds(c * 128 + k * 16, 16)]
                    for m in _MOD_STEPS[i]:
                        q = jnp.int32(m * p)
                        v = jnp.where(v >= q, v - q, v)
                    iv[c, pl.ds(k * 16, 16)] = v
            copies = [
                pltpu.async_copy(
                    tables[i].at[iv.at[c]],
                    rows.at[pl.ds(c * 128, 128)],
                    sem,
                )
                for c in range(nch)
            ]
            for cp in copies:
                cp.wait()
            pltpu.sync_copy(rows, out_hbm.at[i, pl.ds(base, npw), :])

    return gk(*tabs, ti)


def _pad_body(t_ref, o_ref):
    o_ref[:, :SUB] = t_ref[...].astype(jnp.bfloat16)
    o_ref[:, SUB:] = jnp.zeros((t_ref.shape[0], WPAD - SUB), jnp.bfloat16)


def _tc_pad(tab):
    p = tab.shape[0]
    rb = p // 10
    return pl.pallas_call(
        _pad_body,
        grid=(p // rb,),
        in_specs=[pl.BlockSpec((rb, SUB), lambda i: (i, 0))],
        out_specs=pl.BlockSpec((rb, WPAD), lambda i: (i, 0)),
        out_shape=jax.ShapeDtypeStruct((p, WPAD), jnp.bfloat16),
    )(tab)


def _tc_body(x_ref, g_ref, pe_ref, we_ref, wt_ref, b_ref, o_ref):
    acc = jnp.dot(x_ref[...], we_ref[...], preferred_element_type=jnp.float32)
    acc += pe_ref[...] + b_ref[...]
    for i in range(len(PERIODS)):
        acc += jnp.dot(g_ref[i], wt_ref[i], preferred_element_type=jnp.float32)
    o_ref[...] = acc


def _tc_combine(xf, g, pe, we, wt, bias, tb):
    bt, c = xf.shape
    t = pe.shape[0]
    jblocks = t // tb
    b = bt // t
    grid = (jblocks, b)
    return pl.pallas_call(
        _tc_body,
        grid=grid,
        in_specs=[
            pl.BlockSpec((tb, c), lambda j, bb: (bb * jblocks + j, 0)),
            pl.BlockSpec((len(PERIODS), tb, WPAD),
                         lambda j, bb: (0, bb * jblocks + j, 0)),
            pl.BlockSpec((tb, D_MODEL), lambda j, bb: (j, 0)),
            pl.BlockSpec((c, D_MODEL), lambda j, bb: (0, 0)),
            pl.BlockSpec((len(PERIODS), WPAD, D_MODEL), lambda j, bb: (0, 0, 0)),
            pl.BlockSpec((1, D_MODEL), lambda j, bb: (0, 0)),
        ],
        out_specs=pl.BlockSpec((tb, D_MODEL), lambda j, bb: (bb * jblocks + j, 0)),
        out_shape=jax.ShapeDtypeStruct((bt, D_MODEL), jnp.float32),
    )(xf, g, pe, we, wt, bias)


def kernel(x, time_indices, value_W, value_b, daily_tab0, daily_tab1,
           daily_tab2, daily_tab3, daily_tab4, daily_W, daily_b, phase_embed,
           pos_W, pos_b, neg_W, neg_b, vq_W, vq_b, vq_cW, vq_cb):
    B, T, C = x.shape
    bt = B * T
    ti = time_indices.reshape(bt).astype(jnp.int32)
    tabs = [daily_tab0, daily_tab1, daily_tab2, daily_tab3, daily_tab4]
    tabs = [_tc_pad(t) for t in tabs]

    g = _sc_gather(ti, tabs)

    dq = vq_W.shape[0]
    w_q = vq_cW[:, :dq] @ vq_W[:, 0]
    b_q = vq_cW[:, :dq] @ vq_b + vq_cb
    neg_perm = jnp.stack([neg_W[:, 0], neg_W[:, 2], neg_W[:, 1]], axis=1)
    w_eff = value_W + pos_W + 0.1 * neg_perm + (w_q / 660.0)[:, None]
    bias = (value_b + daily_b + pos_b + 0.1 * neg_b + phase_embed.mean(0)
            + b_q - w_q)

    pe = jnp.asarray(_PE[:T])
    wt = jnp.pad(daily_W.T.reshape(len(PERIODS), SUB, D_MODEL),
                 ((0, 0), (0, WPAD - SUB), (0, 0))).astype(jnp.bfloat16)

    out = _tc_combine(x.reshape(bt, C), g, pe, w_eff.T, wt,
                      bias.reshape(1, D_MODEL), tb=512)
    return out.reshape(B, T, D_MODEL)


# speedup vs baseline: 1.9764x; 1.9407x over previous
import functools

import jax
import jax.numpy as jnp
import numpy as np
from jax import lax
from jax.experimental import pallas as pl
from jax.experimental.pallas import tpu as pltpu
from jax.experimental.pallas import tpu_sc as plsc

D_MODEL = 512
SPD = 86400
PERIODS = [SPD, SPD // 2, SPD // 3, SPD // 4, SPD // 6]
SUB = D_MODEL // len(PERIODS)
_MOD_STEPS = [(), (1,), (2, 1), (2, 1), (4, 2, 1)]


def _pe_table(d_model=D_MODEL, max_len=5000, power_freq=50.0, sample_rate=1.0):
    pe = np.zeros((max_len, d_model), dtype=np.float32)
    pos = np.arange(max_len, dtype=np.float32)
    harmonics = [1, 2, 3, 5, 7]
    hd = d_model // (len(harmonics) * 2)
    for h_idx, h in enumerate(harmonics):
        omega = 2.0 * np.pi * power_freq * h / sample_rate
        start = h_idx * hd * 2
        end = min(start + hd * 2, d_model)
        for i in range(0, end - start, 2):
            ps = i * np.pi / (end - start)
            if start + i < d_model:
                pe[:, start + i] = np.sin(pos * omega + ps)
            if start + i + 1 < d_model:
                pe[:, start + i + 1] = np.cos(pos * omega + ps)
    return pe


_PE = _pe_table()


WPAD = 128


def _sc_gather(ti, tabs):
    (bt,) = ti.shape
    info = plsc.get_sparse_core_info()
    nc, ns = info.num_cores, info.num_subcores
    nw = nc * ns
    npw = bt // nw
    nch = npw // 128
    assert npw % 128 == 0

    mesh = plsc.VectorSubcoreMesh(core_axis_name="c", subcore_axis_name="s")

    @functools.partial(
        pl.kernel,
        mesh=mesh,
        compiler_params=pltpu.CompilerParams(use_tc_tiling_on_sc=False),
        out_type=jax.ShapeDtypeStruct((len(tabs), bt, WPAD), jnp.float32),
        scratch_types=[
            pltpu.VMEM((npw,), jnp.int32),
            pltpu.VMEM((nch, 128), jnp.int32),
            pltpu.VMEM((npw, WPAD), jnp.float32),
            pltpu.SemaphoreType.DMA,
        ],
    )
    def gk(t0, t1, t2, t3, t4, ti_hbm, out_hbm, tv, iv, rows, sem):
        wid = lax.axis_index("s") * nc + lax.axis_index("c")
        base = wid * npw
        pltpu.sync_copy(ti_hbm.at[pl.ds(base, npw)], tv)
        tables = [t0, t1, t2, t3, t4]
        for i in range(len(tables)):
            p = PERIODS[i]
            for c in range(nch):
                for k in range(128 // 16):
                    v = tv[pl.ds(c * 128 + k * 16, 16)]
                    for m in _MOD_STEPS[i]:
                        q = jnp.int32(m * p)
                        v = jnp.where(v >= q, v - q, v)
                    iv[c, pl.ds(k * 16, 16)] = v
            copies = [
                pltpu.async_copy(
                    tables[i].at[iv.at[c]],
                    rows.at[pl.ds(c * 128, 128)],
                    sem,
                )
                for c in range(nch)
            ]
            for cp in copies:
                cp.wait()
            pltpu.sync_copy(rows, out_hbm.at[i, pl.ds(base, npw), :])

    return gk(*tabs, ti)


def _pad_body(t_ref, o_ref):
    o_ref[...] = jnp.concatenate(
        [t_ref[...],
         jnp.zeros((t_ref.shape[0], WPAD - SUB), jnp.float32)], axis=1)


def _tc_pad(tab):
    p = tab.shape[0]
    rb = p // 5
    return pl.pallas_call(
        _pad_body,
        grid=(p // rb,),
        in_specs=[pl.BlockSpec((rb, SUB), lambda i: (i, 0))],
        out_specs=pl.BlockSpec((rb, WPAD), lambda i: (i, 0)),
        out_shape=jax.ShapeDtypeStruct((p, WPAD), jnp.float32),
    )(tab)


def _tc_body(x_ref, g_ref, pe_ref, we_ref, wt_ref, b_ref, o_ref):
    acc = jnp.dot(x_ref[...], we_ref[...], preferred_element_type=jnp.float32)
    acc += pe_ref[...] + b_ref[...]
    for i in range(len(PERIODS)):
        acc += jnp.dot(g_ref[i].astype(jnp.bfloat16), wt_ref[i],
                       preferred_element_type=jnp.float32)
    o_ref[...] = acc


def _tc_combine(xf, g, pe, we, wt, bias, tb):
    bt, c = xf.shape
    t = pe.shape[0]
    jblocks = t // tb
    b = bt // t
    grid = (jblocks, b)
    return pl.pallas_call(
        _tc_body,
        grid=grid,
        in_specs=[
            pl.BlockSpec((tb, c), lambda j, bb: (bb * jblocks + j, 0)),
            pl.BlockSpec((len(PERIODS), tb, WPAD),
                         lambda j, bb: (0, bb * jblocks + j, 0)),
            pl.BlockSpec((tb, D_MODEL), lambda j, bb: (j, 0)),
            pl.BlockSpec((c, D_MODEL), lambda j, bb: (0, 0)),
            pl.BlockSpec((len(PERIODS), WPAD, D_MODEL), lambda j, bb: (0, 0, 0)),
            pl.BlockSpec((1, D_MODEL), lambda j, bb: (0, 0)),
        ],
        out_specs=pl.BlockSpec((tb, D_MODEL), lambda j, bb: (bb * jblocks + j, 0)),
        out_shape=jax.ShapeDtypeStruct((bt, D_MODEL), jnp.float32),
    )(xf, g, pe, we, wt, bias)


def kernel(x, time_indices, value_W, value_b, daily_tab0, daily_tab1,
           daily_tab2, daily_tab3, daily_tab4, daily_W, daily_b, phase_embed,
           pos_W, pos_b, neg_W, neg_b, vq_W, vq_b, vq_cW, vq_cb):
    B, T, C = x.shape
    bt = B * T
    ti = time_indices.reshape(bt).astype(jnp.int32)
    tabs = [daily_tab0, daily_tab1, daily_tab2, daily_tab3, daily_tab4]
    tabs = [_tc_pad(t) for t in tabs]

    g = _sc_gather(ti, tabs)

    dq = vq_W.shape[0]
    w_q = vq_cW[:, :dq] @ vq_W[:, 0]
    b_q = vq_cW[:, :dq] @ vq_b + vq_cb
    neg_perm = jnp.stack([neg_W[:, 0], neg_W[:, 2], neg_W[:, 1]], axis=1)
    w_eff = value_W + pos_W + 0.1 * neg_perm + (w_q / 660.0)[:, None]
    bias = (value_b + daily_b + pos_b + 0.1 * neg_b + phase_embed.mean(0)
            + b_q - w_q)

    pe = jnp.asarray(_PE[:T])
    wt = jnp.pad(daily_W.T.reshape(len(PERIODS), SUB, D_MODEL),
                 ((0, 0), (0, WPAD - SUB), (0, 0))).astype(jnp.bfloat16)

    out = _tc_combine(x.reshape(bt, C), g, pe, w_eff.T, wt,
                      bias.reshape(1, D_MODEL), tb=512)
    return out.reshape(B, T, D_MODEL)


# speedup vs baseline: 2.0526x; 1.0386x over previous
import functools

import jax
import jax.numpy as jnp
import numpy as np
from jax import lax
from jax.experimental import pallas as pl
from jax.experimental.pallas import tpu as pltpu
from jax.experimental.pallas import tpu_sc as plsc

D_MODEL = 512
SPD = 86400
PERIODS = [SPD, SPD // 2, SPD // 3, SPD // 4, SPD // 6]
SUB = D_MODEL // len(PERIODS)
_MOD_STEPS = [(), (1,), (2, 1), (2, 1), (4, 2, 1)]


def _pe_table(d_model=D_MODEL, max_len=5000, power_freq=50.0, sample_rate=1.0):
    pe = np.zeros((max_len, d_model), dtype=np.float32)
    pos = np.arange(max_len, dtype=np.float32)
    harmonics = [1, 2, 3, 5, 7]
    hd = d_model // (len(harmonics) * 2)
    for h_idx, h in enumerate(harmonics):
        omega = 2.0 * np.pi * power_freq * h / sample_rate
        start = h_idx * hd * 2
        end = min(start + hd * 2, d_model)
        for i in range(0, end - start, 2):
            ps = i * np.pi / (end - start)
            if start + i < d_model:
                pe[:, start + i] = np.sin(pos * omega + ps)
            if start + i + 1 < d_model:
                pe[:, start + i + 1] = np.cos(pos * omega + ps)
    return pe


_PE = _pe_table()


WPAD = 128


def _sc_gather(ti, tabs):
    (bt,) = ti.shape
    info = plsc.get_sparse_core_info()
    nc, ns = info.num_cores, info.num_subcores
    nw = nc * ns
    npw = bt // nw
    nch = npw // 128
    assert npw % 128 == 0

    mesh = plsc.VectorSubcoreMesh(core_axis_name="c", subcore_axis_name="s")

    @functools.partial(
        pl.kernel,
        mesh=mesh,
        compiler_params=pltpu.CompilerParams(use_tc_tiling_on_sc=False),
        out_type=jax.ShapeDtypeStruct((len(tabs), bt, WPAD), jnp.float32),
        scratch_types=[
            pltpu.VMEM((npw,), jnp.int32),
            pltpu.VMEM((nch, 128), jnp.int32),
            pltpu.VMEM((npw, WPAD), jnp.float32),
            pltpu.SemaphoreType.DMA,
        ],
    )
    def gk(t0, t1, t2, t3, t4, ti_hbm, out_hbm, tv, iv, rows, sem):
        wid = lax.axis_index("s") * nc + lax.axis_index("c")
        base = wid * npw
        pltpu.sync_copy(ti_hbm.at[pl.ds(base, npw)], tv)
        tables = [t0, t1, t2, t3, t4]
        for i in range(len(tables)):
            p = PERIODS[i]
            for c in range(nch):
                for k in range(128 // 16):
                    v = tv[pl.ds(c * 128 + k * 16, 16)]
                    for m in _MOD_STEPS[i]:
                        q = jnp.int32(m * p)
                        v = jnp.where(v >= q, v - q, v)
                    iv[c, pl.ds(k * 16, 16)] = v
            copies = [
                pltpu.async_copy(
                    tables[i].at[iv.at[c]],
                    rows.at[pl.ds(c * 128, 128)],
                    sem,
                )
                for c in range(nch)
            ]
            for cp in copies:
                cp.wait()
            pltpu.sync_copy(rows, out_hbm.at[i, pl.ds(base, npw), :])

    return gk(*tabs, ti)


def _pad_body(t_ref, o_ref):
    o_ref[...] = jnp.concatenate(
        [t_ref[...],
         jnp.zeros((t_ref.shape[0], WPAD - SUB), jnp.float32)], axis=1)


def _tc_pad(tab):
    p = tab.shape[0]
    rb = p // 5
    return pl.pallas_call(
        _pad_body,
        grid=(p // rb,),
        in_specs=[pl.BlockSpec((rb, SUB), lambda i: (i, 0))],
        out_specs=pl.BlockSpec((rb, WPAD), lambda i: (i, 0)),
        out_shape=jax.ShapeDtypeStruct((p, WPAD), jnp.float32),
    )(tab)


def _tc_body(x_ref, g_ref, pe_ref, we_ref, wt_ref, b_ref, o_ref):
    acc = jnp.dot(x_ref[...], we_ref[...], preferred_element_type=jnp.float32)
    acc += pe_ref[...] + b_ref[...]
    for i in range(len(PERIODS)):
        acc += jnp.dot(g_ref[i].astype(jnp.bfloat16), wt_ref[i],
                       preferred_element_type=jnp.float32)
    o_ref[...] = acc


def _tc_combine(xf, g, pe, we, wt, bias, tb):
    bt, c = xf.shape
    t = pe.shape[0]
    jblocks = t // tb
    b = bt // t
    grid = (jblocks, b)
    return pl.pallas_call(
        _tc_body,
        grid=grid,
        in_specs=[
            pl.BlockSpec((tb, c), lambda j, bb: (bb * jblocks + j, 0)),
            pl.BlockSpec((len(PERIODS), tb, WPAD),
                         lambda j, bb: (0, bb * jblocks + j, 0)),
            pl.BlockSpec((tb, D_MODEL), lambda j, bb: (j, 0)),
            pl.BlockSpec((c, D_MODEL), lambda j, bb: (0, 0)),
            pl.BlockSpec((len(PERIODS), WPAD, D_MODEL), lambda j, bb: (0, 0, 0)),
            pl.BlockSpec((1, D_MODEL), lambda j, bb: (0, 0)),
        ],
        out_specs=pl.BlockSpec((tb, D_MODEL), lambda j, bb: (bb * jblocks + j, 0)),
        out_shape=jax.ShapeDtypeStruct((bt, D_MODEL), jnp.float32),
    )(xf, g, pe, we, wt, bias)


def kernel(x, time_indices, value_W, value_b, daily_tab0, daily_tab1,
           daily_tab2, daily_tab3, daily_tab4, daily_W, daily_b, phase_embed,
           pos_W, pos_b, neg_W, neg_b, vq_W, vq_b, vq_cW, vq_cb):
    B, T, C = x.shape
    bt = B * T
    ti = time_indices.reshape(bt).astype(jnp.int32)
    tabs = [daily_tab0, daily_tab1, daily_tab2, daily_tab3, daily_tab4]
    tabs = [_tc_pad(t) for t in tabs]

    g = _sc_gather(ti, tabs)

    dq = vq_W.shape[0]
    w_q = vq_cW[:, :dq] @ vq_W[:, 0]
    b_q = vq_cW[:, :dq] @ vq_b + vq_cb
    neg_perm = jnp.stack([neg_W[:, 0], neg_W[:, 2], neg_W[:, 1]], axis=1)
    w_eff = value_W + pos_W + 0.1 * neg_perm + (w_q / 660.0)[:, None]
    bias = (value_b + daily_b + pos_b + 0.1 * neg_b + phase_embed.mean(0)
            + b_q - w_q)

    pe = jnp.asarray(_PE[:T])
    wt = jnp.pad(daily_W.T.reshape(len(PERIODS), SUB, D_MODEL),
                 ((0, 0), (0, WPAD - SUB), (0, 0))).astype(jnp.bfloat16)

    out = _tc_combine(x.reshape(bt, C), g, pe, w_eff.T, wt,
                      bias.reshape(1, D_MODEL), tb=1024)
    return out.reshape(B, T, D_MODEL)


# speedup vs baseline: 2.1577x; 1.0512x over previous
import functools

import jax
import jax.numpy as jnp
import numpy as np
from jax import lax
from jax.experimental import pallas as pl
from jax.experimental.pallas import tpu as pltpu
from jax.experimental.pallas import tpu_sc as plsc

D_MODEL = 512
SPD = 86400
PERIODS = [SPD, SPD // 2, SPD // 3, SPD // 4, SPD // 6]
SUB = D_MODEL // len(PERIODS)
_MOD_STEPS = [(), (1,), (2, 1), (2, 1), (4, 2, 1)]


def _pe_table(d_model=D_MODEL, max_len=5000, power_freq=50.0, sample_rate=1.0):
    pe = np.zeros((max_len, d_model), dtype=np.float32)
    pos = np.arange(max_len, dtype=np.float32)
    harmonics = [1, 2, 3, 5, 7]
    hd = d_model // (len(harmonics) * 2)
    for h_idx, h in enumerate(harmonics):
        omega = 2.0 * np.pi * power_freq * h / sample_rate
        start = h_idx * hd * 2
        end = min(start + hd * 2, d_model)
        for i in range(0, end - start, 2):
            ps = i * np.pi / (end - start)
            if start + i < d_model:
                pe[:, start + i] = np.sin(pos * omega + ps)
            if start + i + 1 < d_model:
                pe[:, start + i + 1] = np.cos(pos * omega + ps)
    return pe


_PE = _pe_table()


WPAD = 128


def _sc_gather1(ti, tab, i):
    (bt,) = ti.shape
    info = plsc.get_sparse_core_info()
    nc, ns = info.num_cores, info.num_subcores
    nw = nc * ns
    npw = bt // nw
    nch = npw // 128
    assert npw % 128 == 0

    mesh = plsc.VectorSubcoreMesh(core_axis_name="c", subcore_axis_name="s")

    @functools.partial(
        pl.kernel,
        mesh=mesh,
        compiler_params=pltpu.CompilerParams(use_tc_tiling_on_sc=False),
        out_type=jax.ShapeDtypeStruct((bt, WPAD), jnp.float32),
        scratch_types=[
            pltpu.VMEM((npw,), jnp.int32),
            pltpu.VMEM((nch, 128), jnp.int32),
            pltpu.VMEM((npw, WPAD), jnp.float32),
            pltpu.SemaphoreType.DMA,
        ],
    )
    def gk(tab_hbm, ti_hbm, out_hbm, tv, iv, rows, sem):
        wid = lax.axis_index("s") * nc + lax.axis_index("c")
        base = wid * npw
        pltpu.sync_copy(ti_hbm.at[pl.ds(base, npw)], tv)
        p = PERIODS[i]
        for c in range(nch):
            for k in range(128 // 16):
                v = tv[pl.ds(c * 128 + k * 16, 16)]
                for m in _MOD_STEPS[i]:
                    q = jnp.int32(m * p)
                    v = jnp.where(v >= q, v - q, v)
                iv[c, pl.ds(k * 16, 16)] = v
        copies = [
            pltpu.async_copy(
                tab_hbm.at[iv.at[c]],
                rows.at[pl.ds(c * 128, 128)],
                sem,
            )
            for c in range(nch)
        ]
        for cp in copies:
            cp.wait()
        pltpu.sync_copy(rows, out_hbm.at[pl.ds(base, npw), :])

    return gk(tab, ti)


def _pad_body(t_ref, o_ref):
    o_ref[...] = jnp.concatenate(
        [t_ref[...],
         jnp.zeros((t_ref.shape[0], WPAD - SUB), jnp.float32)], axis=1)


def _tc_pad(tab):
    p = tab.shape[0]
    rb = p // 5
    return pl.pallas_call(
        _pad_body,
        grid=(p // rb,),
        in_specs=[pl.BlockSpec((rb, SUB), lambda i: (i, 0))],
        out_specs=pl.BlockSpec((rb, WPAD), lambda i: (i, 0)),
        out_shape=jax.ShapeDtypeStruct((p, WPAD), jnp.float32),
    )(tab)


def _tc_body(x_ref, g0, g1, g2, g3, g4, pe_ref, we_ref, wt_ref, b_ref,
             o_ref):
    acc = jnp.dot(x_ref[...], we_ref[...], preferred_element_type=jnp.float32)
    acc += pe_ref[...] + b_ref[...]
    for i, g_ref in enumerate([g0, g1, g2, g3, g4]):
        acc += jnp.dot(g_ref[...].astype(jnp.bfloat16), wt_ref[i],
                       preferred_element_type=jnp.float32)
    o_ref[...] = acc


def _tc_combine(xf, gs, pe, we, wt, bias, tb):
    bt, c = xf.shape
    t = pe.shape[0]
    jblocks = t // tb
    b = bt // t
    grid = (jblocks, b)
    tok = lambda j, bb: (bb * jblocks + j, 0)
    return pl.pallas_call(
        _tc_body,
        grid=grid,
        in_specs=[
            pl.BlockSpec((tb, c), tok),
            pl.BlockSpec((tb, WPAD), tok),
            pl.BlockSpec((tb, WPAD), tok),
            pl.BlockSpec((tb, WPAD), tok),
            pl.BlockSpec((tb, WPAD), tok),
            pl.BlockSpec((tb, WPAD), tok),
            pl.BlockSpec((tb, D_MODEL), lambda j, bb: (j, 0)),
            pl.BlockSpec((c, D_MODEL), lambda j, bb: (0, 0)),
            pl.BlockSpec((len(PERIODS), WPAD, D_MODEL), lambda j, bb: (0, 0, 0)),
            pl.BlockSpec((1, D_MODEL), lambda j, bb: (0, 0)),
        ],
        out_specs=pl.BlockSpec((tb, D_MODEL), tok),
        out_shape=jax.ShapeDtypeStruct((bt, D_MODEL), jnp.float32),
    )(xf, *gs, pe, we, wt, bias)


def kernel(x, time_indices, value_W, value_b, daily_tab0, daily_tab1,
           daily_tab2, daily_tab3, daily_tab4, daily_W, daily_b, phase_embed,
           pos_W, pos_b, neg_W, neg_b, vq_W, vq_b, vq_cW, vq_cb):
    B, T, C = x.shape
    bt = B * T
    ti = time_indices.reshape(bt).astype(jnp.int32)
    tabs = [daily_tab0, daily_tab1, daily_tab2, daily_tab3, daily_tab4]
    gs = [_sc_gather1(ti, _tc_pad(t), i) for i, t in enumerate(tabs)]

    dq = vq_W.shape[0]
    w_q = vq_cW[:, :dq] @ vq_W[:, 0]
    b_q = vq_cW[:, :dq] @ vq_b + vq_cb
    neg_perm = jnp.stack([neg_W[:, 0], neg_W[:, 2], neg_W[:, 1]], axis=1)
    w_eff = value_W + pos_W + 0.1 * neg_perm + (w_q / 660.0)[:, None]
    bias = (value_b + daily_b + pos_b + 0.1 * neg_b + phase_embed.mean(0)
            + b_q - w_q)

    pe = jnp.asarray(_PE[:T])
    wt = jnp.pad(daily_W.T.reshape(len(PERIODS), SUB, D_MODEL),
                 ((0, 0), (0, WPAD - SUB), (0, 0))).astype(jnp.bfloat16)

    out = _tc_combine(x.reshape(bt, C), gs, pe, w_eff.T, wt,
                      bias.reshape(1, D_MODEL), tb=1024)
    return out.reshape(B, T, D_MODEL)
